# Initial kernel scaffold; baseline (speedup 1.0000x reference)
import jax
import jax.numpy as jnp
from jax.experimental import pallas as pl


def kernel(x, edge_index, batch, W_embed, b_embed, Ws, bs, gammas, betas, W_o1, b_o1, W_o2, b_o2):
    raise NotImplementedError("write your pallas kernel here")


# speedup vs baseline: 11.3952x; 11.3952x over previous
import functools

import jax
import jax.numpy as jnp
from jax import lax
from jax.experimental import pallas as pl
from jax.experimental.pallas import tpu as pltpu
from jax.experimental.pallas import tpu_sc as plsc

N = 50000
H = 64
HC = 32
G = 16
OUT = 128
L = 4

NC = 2
NS = 16
B = 128
E_PAD = 819200
NB = E_PAD // B
S_ROWS = 51200
RT = S_ROWS // NS

DUMMY_DST = N

ROW_BLK = 2000


def _fill_f32(ref, rows, cols, val):
    v = jnp.full((16,), val, jnp.float32)

    def body(i, _):
        for c0 in range(0, cols, 16):
            ref[i, c0:c0 + 16] = v
        return 0

    lax.fori_loop(0, rows, body, 0)


def _sc_degree(col2):
    nbt = NB // (NC * NS)
    mesh = plsc.VectorSubcoreMesh(core_axis_name="c", subcore_axis_name="s")

    @functools.partial(
        pl.kernel,
        mesh=mesh,
        compiler_params=pltpu.CompilerParams(use_tc_tiling_on_sc=False),
        out_type=[
            jax.ShapeDtypeStruct((S_ROWS, 16), jnp.float32),
            jax.ShapeDtypeStruct((S_ROWS, 16), jnp.float32),
        ],
        scratch_types=[
            pltpu.VMEM((nbt, B), jnp.int32),
            pltpu.VMEM((B, 16), jnp.float32),
            pltpu.VMEM((B, 16), jnp.float32),
            pltpu.VMEM_SHARED((S_ROWS, 16), jnp.float32),
        ],
    )
    def k(col_hbm, d0_hbm, d1_hbm, cidx_v, ones_v, zero_v, deg_sh):
        cid = lax.axis_index("c")
        sid = lax.axis_index("s")
        wid = sid * NC + cid

        _fill_f32(ones_v, B, 16, 1.0)
        _fill_f32(zero_v, B, 16, 0.0)

        def zbody(j, _):
            pltpu.sync_copy(zero_v, deg_sh.at[pl.ds(sid * RT + j * B, B)])
            return 0
        lax.fori_loop(0, RT // B, zbody, 0)

        pltpu.sync_copy(col_hbm.at[pl.ds(wid * nbt, nbt)], cidx_v)

        plsc.subcore_barrier()

        def sbody(g, _):
            pltpu.sync_copy(ones_v, deg_sh.at[cidx_v.at[g]], add=True)
            return 0
        lax.fori_loop(0, nbt, sbody, 0)

        plsc.subcore_barrier()

        @pl.when(cid == 0)
        def _():
            pltpu.sync_copy(deg_sh.at[pl.ds(sid * RT, RT)],
                            d0_hbm.at[pl.ds(sid * RT, RT)])

        @pl.when(cid == 1)
        def _():
            pltpu.sync_copy(deg_sh.at[pl.ds(sid * RT, RT)],
                            d1_hbm.at[pl.ds(sid * RT, RT)])

    return k(col2)


def _sc_layer(row2, col2, hws_a, hws_b):
    nbt = NB // NS
    mesh = plsc.VectorSubcoreMesh(core_axis_name="c", subcore_axis_name="s")

    KI = 8

    @functools.partial(
        pl.kernel,
        mesh=mesh,
        compiler_params=pltpu.CompilerParams(use_tc_tiling_on_sc=False),
        out_type=[
            jax.ShapeDtypeStruct((S_ROWS, HC), jnp.float32),
            jax.ShapeDtypeStruct((S_ROWS, HC), jnp.float32),
        ],
        scratch_types=[
            pltpu.VMEM((KI, B), jnp.int32),
            pltpu.VMEM((KI, B), jnp.int32),
            pltpu.VMEM((2 * B, HC), jnp.float32),
            pltpu.VMEM_SHARED((S_ROWS, HC), jnp.float32),
            pltpu.SemaphoreType.DMA,
            pltpu.SemaphoreType.DMA,
        ],
    )
    def k(row_hbm, col_hbm, ha_hbm, hb_hbm, s0_hbm, s1_hbm,
          ridx_v, cidx_v, rows_v, s_sh, sem0, sem1):
        cid = lax.axis_index("c")
        sid = lax.axis_index("s")

        _fill_f32(rows_v, 2 * B, HC, 0.0)

        def zbody(j, _):
            pltpu.sync_copy(rows_v.at[pl.ds(0, B)],
                            s_sh.at[pl.ds(sid * RT + j * B, B)])
            return 0
        lax.fori_loop(0, RT // B, zbody, 0)

        plsc.subcore_barrier()

        def run(tab_hbm):
            def chunk(g, _):
                base = sid * nbt + g * KI
                pltpu.sync_copy(row_hbm.at[pl.ds(base, KI)], ridx_v)
                pltpu.sync_copy(col_hbm.at[pl.ds(base, KI)], cidx_v)
                for j in range(0, KI, 2):
                    c0 = pltpu.async_copy(tab_hbm.at[ridx_v.at[j]],
                                          rows_v.at[pl.ds(0, B)], sem0)
                    c1 = pltpu.async_copy(tab_hbm.at[ridx_v.at[j + 1]],
                                          rows_v.at[pl.ds(B, B)], sem1)
                    c0.wait()
                    pltpu.sync_copy(rows_v.at[pl.ds(0, B)],
                                    s_sh.at[cidx_v.at[j]], add=True)
                    c1.wait()
                    pltpu.sync_copy(rows_v.at[pl.ds(B, B)],
                                    s_sh.at[cidx_v.at[j + 1]], add=True)
                return 0
            lax.fori_loop(0, nbt // KI, chunk, 0)

        @pl.when(cid == 0)
        def _():
            run(ha_hbm)

        @pl.when(cid == 1)
        def _():
            run(hb_hbm)

        plsc.subcore_barrier()

        @pl.when(cid == 0)
        def _():
            pltpu.sync_copy(s_sh.at[pl.ds(sid * RT, RT)],
                            s0_hbm.at[pl.ds(sid * RT, RT)])

        @pl.when(cid == 1)
        def _():
            pltpu.sync_copy(s_sh.at[pl.ds(sid * RT, RT)],
                            s1_hbm.at[pl.ds(sid * RT, RT)])

    return k(row2, col2, hws_a, hws_b)


def _tc_pre_body(x_ref, d0_ref, d1_ref, we_ref, be_ref, w0_ref,
                 h_ref, dis_ref, ha_ref, hb_ref):
    xb = x_ref[...]
    h = jnp.maximum(
        jnp.dot(xb, we_ref[...], preferred_element_type=jnp.float32)
        + be_ref[...], 0.0)
    deg = d0_ref[:, 0:1] + d1_ref[:, 0:1] + 1.0
    dis = lax.rsqrt(deg)
    hws = dis * jnp.dot(h, w0_ref[...], preferred_element_type=jnp.float32)
    h_ref[...] = h
    dis_ref[...] = dis
    ha_ref[...] = hws[:, :HC]
    hb_ref[...] = hws[:, HC:]


def _tc_pre(x, d0, d1, We, be, W0):
    grid = (N // ROW_BLK,)
    return pl.pallas_call(
        _tc_pre_body,
        grid=grid,
        in_specs=[
            pl.BlockSpec((ROW_BLK, 9), lambda i: (i, 0)),
            pl.BlockSpec((ROW_BLK, 16), lambda i: (i, 0)),
            pl.BlockSpec((ROW_BLK, 16), lambda i: (i, 0)),
            pl.BlockSpec((9, H), lambda i: (0, 0)),
            pl.BlockSpec((1, H), lambda i: (0, 0)),
            pl.BlockSpec((H, H), lambda i: (0, 0)),
        ],
        out_specs=[
            pl.BlockSpec((ROW_BLK, H), lambda i: (i, 0)),
            pl.BlockSpec((ROW_BLK, 1), lambda i: (i, 0)),
            pl.BlockSpec((ROW_BLK, HC), lambda i: (i, 0)),
            pl.BlockSpec((ROW_BLK, HC), lambda i: (i, 0)),
        ],
        out_shape=[
            jax.ShapeDtypeStruct((N, H), jnp.float32),
            jax.ShapeDtypeStruct((N, 1), jnp.float32),
            jax.ShapeDtypeStruct((N, HC), jnp.float32),
            jax.ShapeDtypeStruct((N, HC), jnp.float32),
        ],
    )(x, d0, d1, We, be, W0)


def _layer_update(h_ref, dis_ref, s0_ref, s1_ref, ha_ref, hb_ref,
                  b_ref, g_ref, bt_ref):
    S = jnp.concatenate([s0_ref[...], s1_ref[...]], axis=1)
    hws = jnp.concatenate([ha_ref[...], hb_ref[...]], axis=1)
    dis = dis_ref[...]
    agg = dis * (S + hws) + b_ref[...]
    mu = jnp.mean(agg, axis=1, keepdims=True)
    diff = agg - mu
    var = jnp.mean(diff * diff, axis=1, keepdims=True)
    hn = diff * lax.rsqrt(var + 1e-5) * g_ref[...] + bt_ref[...]
    return h_ref[...] + jnp.maximum(hn, 0.0), dis


def _tc_layer_body(h_ref, dis_ref, s0_ref, s1_ref, ha_ref, hb_ref,
                   b_ref, g_ref, bt_ref, wn_ref,
                   ho_ref, hao_ref, hbo_ref):
    h_new, dis = _layer_update(h_ref, dis_ref, s0_ref, s1_ref, ha_ref,
                               hb_ref, b_ref, g_ref, bt_ref)
    ho_ref[...] = h_new
    hws = dis * jnp.dot(h_new, wn_ref[...], preferred_element_type=jnp.float32)
    hao_ref[...] = hws[:, :HC]
    hbo_ref[...] = hws[:, HC:]


def _tc_layer(h, dis, s0, s1, ha, hb, b, g, bt, Wn):
    grid = (N // ROW_BLK,)
    rb = lambda i: (i, 0)
    z = lambda i: (0, 0)
    return pl.pallas_call(
        _tc_layer_body,
        grid=grid,
        in_specs=[
            pl.BlockSpec((ROW_BLK, H), rb),
            pl.BlockSpec((ROW_BLK, 1), rb),
            pl.BlockSpec((ROW_BLK, HC), rb),
            pl.BlockSpec((ROW_BLK, HC), rb),
            pl.BlockSpec((ROW_BLK, HC), rb),
            pl.BlockSpec((ROW_BLK, HC), rb),
            pl.BlockSpec((1, H), z),
            pl.BlockSpec((1, H), z),
            pl.BlockSpec((1, H), z),
            pl.BlockSpec((H, H), z),
        ],
        out_specs=[
            pl.BlockSpec((ROW_BLK, H), rb),
            pl.BlockSpec((ROW_BLK, HC), rb),
            pl.BlockSpec((ROW_BLK, HC), rb),
        ],
        out_shape=[
            jax.ShapeDtypeStruct((N, H), jnp.float32),
            jax.ShapeDtypeStruct((N, HC), jnp.float32),
            jax.ShapeDtypeStruct((N, HC), jnp.float32),
        ],
    )(h, dis, s0, s1, ha, hb, b, g, bt, Wn)


def _tc_final_body(h_ref, dis_ref, s0_ref, s1_ref, ha_ref, hb_ref,
                   b_ref, g_ref, bt_ref, batch_ref,
                   wo1_ref, bo1_ref, wo2_ref, bo2_ref,
                   out_ref, pooled_ref, cnt_ref):
    step = pl.program_id(0)
    nsteps = pl.num_programs(0)
    h_new, _ = _layer_update(h_ref, dis_ref, s0_ref, s1_ref, ha_ref,
                             hb_ref, b_ref, g_ref, bt_ref)
    bb = batch_ref[...]
    oh = (bb == lax.broadcasted_iota(jnp.int32, (1, G), 1)).astype(jnp.float32)
    dn = (((0,), (0,)), ((), ()))
    psum = lax.dot_general(oh, h_new, dn, preferred_element_type=jnp.float32)
    csum = lax.dot_general(oh, jnp.ones((oh.shape[0], 1), jnp.float32), dn,
                           preferred_element_type=jnp.float32)

    @pl.when(step == 0)
    def _():
        pooled_ref[...] = psum
        cnt_ref[...] = csum

    @pl.when(step > 0)
    def _():
        pooled_ref[...] += psum
        cnt_ref[...] += csum

    @pl.when(step == nsteps - 1)
    def _():
        pooled = pooled_ref[...] / jnp.maximum(cnt_ref[...], 1.0)
        t = jnp.maximum(
            jnp.dot(pooled, wo1_ref[...], preferred_element_type=jnp.float32)
            + bo1_ref[...], 0.0)
        out_ref[...] = (
            jnp.dot(t, wo2_ref[...], preferred_element_type=jnp.float32)
            + bo2_ref[...])


def _tc_final(h, dis, s0, s1, ha, hb, b, g, bt, batch2,
              Wo1, bo1, Wo2, bo2):
    grid = (N // ROW_BLK,)
    rb = lambda i: (i, 0)
    z = lambda i: (0, 0)
    return pl.pallas_call(
        _tc_final_body,
        grid=grid,
        in_specs=[
            pl.BlockSpec((ROW_BLK, H), rb),
            pl.BlockSpec((ROW_BLK, 1), rb),
            pl.BlockSpec((ROW_BLK, HC), rb),
            pl.BlockSpec((ROW_BLK, HC), rb),
            pl.BlockSpec((ROW_BLK, HC), rb),
            pl.BlockSpec((ROW_BLK, HC), rb),
            pl.BlockSpec((1, H), z),
            pl.BlockSpec((1, H), z),
            pl.BlockSpec((1, H), z),
            pl.BlockSpec((ROW_BLK, 1), rb),
            pl.BlockSpec((H, OUT), z),
            pl.BlockSpec((1, OUT), z),
            pl.BlockSpec((OUT, OUT), z),
            pl.BlockSpec((1, OUT), z),
        ],
        out_specs=pl.BlockSpec((G, OUT), z),
        out_shape=jax.ShapeDtypeStruct((G, OUT), jnp.float32),
        scratch_shapes=[
            pltpu.VMEM((G, H), jnp.float32),
            pltpu.VMEM((G, 1), jnp.float32),
        ],
    )(h, dis, s0, s1, ha, hb, b, g, bt, batch2, Wo1, bo1, Wo2, bo2)


def kernel(x, edge_index, batch, W_embed, b_embed, Ws, bs, gammas, betas,
           W_o1, b_o1, W_o2, b_o2):
    E = edge_index.shape[1]
    npad = E_PAD - E
    row = jnp.concatenate(
        [edge_index[0], jnp.zeros((npad,), jnp.int32)]).reshape(NB, B)
    col = jnp.concatenate(
        [edge_index[1], jnp.full((npad,), DUMMY_DST, jnp.int32)]).reshape(NB, B)

    d0, d1 = _sc_degree(col)
    h, dis, ha, hb = _tc_pre(x, d0[:N], d1[:N], W_embed,
                             b_embed.reshape(1, H), Ws[0])
    for l in range(L):
        s0, s1 = _sc_layer(row, col, ha, hb)
        if l < L - 1:
            h, ha, hb = _tc_layer(h, dis, s0[:N], s1[:N], ha, hb,
                                  bs[l].reshape(1, H),
                                  gammas[l].reshape(1, H),
                                  betas[l].reshape(1, H), Ws[l + 1])
        else:
            out = _tc_final(h, dis, s0[:N], s1[:N], ha, hb,
                            bs[l].reshape(1, H),
                            gammas[l].reshape(1, H),
                            betas[l].reshape(1, H),
                            batch.reshape(N, 1),
                            W_o1, b_o1.reshape(1, OUT),
                            W_o2, b_o2.reshape(1, OUT))
    return out


# speedup vs baseline: 15.0386x; 1.3197x over previous
import functools

import jax
import jax.numpy as jnp
from jax import lax
from jax.experimental import pallas as pl
from jax.experimental.pallas import tpu as pltpu
from jax.experimental.pallas import tpu_sc as plsc

N = 50000
H = 64
HC = 32
G = 16
OUT = 128
L = 4

NC = 2
NS = 16
B = 128
E_PAD = 819200
NB = E_PAD // B
S_ROWS = 51200
RT = S_ROWS // NS

DUMMY_DST = N

ROW_BLK = 2000


def _fill_f32(ref, rows, cols, val):
    v = jnp.full((16,), val, jnp.float32)

    def body(i, _):
        for c0 in range(0, cols, 16):
            ref[i, c0:c0 + 16] = v
        return 0

    lax.fori_loop(0, rows, body, 0)


def _sc_degree(col2):
    nbt = NB // (NC * NS)
    mesh = plsc.VectorSubcoreMesh(core_axis_name="c", subcore_axis_name="s")

    @functools.partial(
        pl.kernel,
        mesh=mesh,
        compiler_params=pltpu.CompilerParams(use_tc_tiling_on_sc=False),
        out_type=[
            jax.ShapeDtypeStruct((S_ROWS, 16), jnp.float32),
            jax.ShapeDtypeStruct((S_ROWS, 16), jnp.float32),
        ],
        scratch_types=[
            pltpu.VMEM((nbt, B), jnp.int32),
            pltpu.VMEM((B, 16), jnp.float32),
            pltpu.VMEM((B, 16), jnp.float32),
            pltpu.VMEM_SHARED((S_ROWS, 16), jnp.float32),
        ],
    )
    def k(col_hbm, d0_hbm, d1_hbm, cidx_v, ones_v, zero_v, deg_sh):
        cid = lax.axis_index("c")
        sid = lax.axis_index("s")
        wid = sid * NC + cid

        _fill_f32(ones_v, B, 16, 1.0)
        _fill_f32(zero_v, B, 16, 0.0)

        def zbody(j, _):
            pltpu.sync_copy(zero_v, deg_sh.at[pl.ds(sid * RT + j * B, B)])
            return 0
        lax.fori_loop(0, RT // B, zbody, 0)

        pltpu.sync_copy(col_hbm.at[pl.ds(wid * nbt, nbt)], cidx_v)

        plsc.subcore_barrier()

        def sbody(g, _):
            pltpu.sync_copy(ones_v, deg_sh.at[cidx_v.at[g]], add=True)
            return 0
        lax.fori_loop(0, nbt, sbody, 0)

        plsc.subcore_barrier()

        @pl.when(cid == 0)
        def _():
            pltpu.sync_copy(deg_sh.at[pl.ds(sid * RT, RT)],
                            d0_hbm.at[pl.ds(sid * RT, RT)])

        @pl.when(cid == 1)
        def _():
            pltpu.sync_copy(deg_sh.at[pl.ds(sid * RT, RT)],
                            d1_hbm.at[pl.ds(sid * RT, RT)])

    return k(col2)


def _sc_layer(row2, col2, hws_a, hws_b):
    nbt = NB // NS
    mesh = plsc.VectorSubcoreMesh(core_axis_name="c", subcore_axis_name="s")

    QB = 20
    NBUF = 5
    LOOK = 3

    @functools.partial(
        pl.kernel,
        mesh=mesh,
        compiler_params=pltpu.CompilerParams(use_tc_tiling_on_sc=False),
        out_type=[
            jax.ShapeDtypeStruct((S_ROWS, HC), jnp.float32),
            jax.ShapeDtypeStruct((S_ROWS, HC), jnp.float32),
        ],
        scratch_types=[
            pltpu.VMEM((QB, B), jnp.int32),
            pltpu.VMEM((QB, B), jnp.int32),
            pltpu.VMEM((NBUF * B, HC), jnp.float32),
            pltpu.VMEM_SHARED((S_ROWS, HC), jnp.float32),
        ] + [pltpu.SemaphoreType.DMA] * NBUF,
    )
    def k(row_hbm, col_hbm, ha_hbm, hb_hbm, s0_hbm, s1_hbm,
          ridx_v, cidx_v, rows_v, s_sh, *sems):
        cid = lax.axis_index("c")
        sid = lax.axis_index("s")

        _fill_f32(rows_v, B, HC, 0.0)

        def zbody(j, _):
            pltpu.sync_copy(rows_v.at[pl.ds(0, B)],
                            s_sh.at[pl.ds(sid * RT + j * B, B)])
            return 0
        lax.fori_loop(0, RT // B, zbody, 0)

        plsc.subcore_barrier()

        def run(tab_hbm):
            def buf(b):
                return rows_v.at[pl.ds(b * B, B)]

            def gather(j, b):
                pltpu.async_copy(tab_hbm.at[ridx_v.at[j]], buf(b), sems[b])

            def scatter(j, b):
                pltpu.async_copy(buf(b), s_sh.at[cidx_v.at[j]],
                                 sems[b], add=True)

            def wait(b):
                pltpu.make_async_copy(buf(b), s_sh.at[cidx_v.at[0]],
                                      sems[b]).wait()

            def slot(q, _):
                base = sid * nbt + q * QB
                pltpu.sync_copy(row_hbm.at[pl.ds(base, QB)], ridx_v)
                pltpu.sync_copy(col_hbm.at[pl.ds(base, QB)], cidx_v)
                for j in range(LOOK):
                    gather(j, j % NBUF)
                for j in range(QB):
                    b = j % NBUF
                    wait(b)
                    scatter(j, b)
                    jn = j + LOOK
                    if jn < QB:
                        b2 = jn % NBUF
                        if jn >= NBUF:
                            wait(b2)
                        gather(jn, b2)
                for j in range(QB - NBUF, QB):
                    wait(j % NBUF)
                return 0
            lax.fori_loop(0, nbt // QB, slot, 0)

        @pl.when(cid == 0)
        def _():
            run(ha_hbm)

        @pl.when(cid == 1)
        def _():
            run(hb_hbm)

        plsc.subcore_barrier()

        @pl.when(cid == 0)
        def _():
            pltpu.sync_copy(s_sh.at[pl.ds(sid * RT, RT)],
                            s0_hbm.at[pl.ds(sid * RT, RT)])

        @pl.when(cid == 1)
        def _():
            pltpu.sync_copy(s_sh.at[pl.ds(sid * RT, RT)],
                            s1_hbm.at[pl.ds(sid * RT, RT)])

    return k(row2, col2, hws_a, hws_b)


def _tc_pre_body(x_ref, d0_ref, d1_ref, we_ref, be_ref, w0_ref,
                 h_ref, dis_ref, ha_ref, hb_ref):
    xb = x_ref[...]
    h = jnp.maximum(
        jnp.dot(xb, we_ref[...], preferred_element_type=jnp.float32)
        + be_ref[...], 0.0)
    deg = d0_ref[:, 0:1] + d1_ref[:, 0:1] + 1.0
    dis = lax.rsqrt(deg)
    hws = dis * jnp.dot(h, w0_ref[...], preferred_element_type=jnp.float32)
    h_ref[...] = h
    dis_ref[...] = dis
    ha_ref[...] = hws[:, :HC]
    hb_ref[...] = hws[:, HC:]


def _tc_pre(x, d0, d1, We, be, W0):
    grid = (N // ROW_BLK,)
    return pl.pallas_call(
        _tc_pre_body,
        grid=grid,
        in_specs=[
            pl.BlockSpec((ROW_BLK, 9), lambda i: (i, 0)),
            pl.BlockSpec((ROW_BLK, 16), lambda i: (i, 0)),
            pl.BlockSpec((ROW_BLK, 16), lambda i: (i, 0)),
            pl.BlockSpec((9, H), lambda i: (0, 0)),
            pl.BlockSpec((1, H), lambda i: (0, 0)),
            pl.BlockSpec((H, H), lambda i: (0, 0)),
        ],
        out_specs=[
            pl.BlockSpec((ROW_BLK, H), lambda i: (i, 0)),
            pl.BlockSpec((ROW_BLK, 1), lambda i: (i, 0)),
            pl.BlockSpec((ROW_BLK, HC), lambda i: (i, 0)),
            pl.BlockSpec((ROW_BLK, HC), lambda i: (i, 0)),
        ],
        out_shape=[
            jax.ShapeDtypeStruct((N, H), jnp.float32),
            jax.ShapeDtypeStruct((N, 1), jnp.float32),
            jax.ShapeDtypeStruct((N, HC), jnp.float32),
            jax.ShapeDtypeStruct((N, HC), jnp.float32),
        ],
    )(x, d0, d1, We, be, W0)


def _layer_update(h_ref, dis_ref, s0_ref, s1_ref, ha_ref, hb_ref,
                  b_ref, g_ref, bt_ref):
    S = jnp.concatenate([s0_ref[...], s1_ref[...]], axis=1)
    hws = jnp.concatenate([ha_ref[...], hb_ref[...]], axis=1)
    dis = dis_ref[...]
    agg = dis * (S + hws) + b_ref[...]
    mu = jnp.mean(agg, axis=1, keepdims=True)
    diff = agg - mu
    var = jnp.mean(diff * diff, axis=1, keepdims=True)
    hn = diff * lax.rsqrt(var + 1e-5) * g_ref[...] + bt_ref[...]
    return h_ref[...] + jnp.maximum(hn, 0.0), dis


def _tc_layer_body(h_ref, dis_ref, s0_ref, s1_ref, ha_ref, hb_ref,
                   b_ref, g_ref, bt_ref, wn_ref,
                   ho_ref, hao_ref, hbo_ref):
    h_new, dis = _layer_update(h_ref, dis_ref, s0_ref, s1_ref, ha_ref,
                               hb_ref, b_ref, g_ref, bt_ref)
    ho_ref[...] = h_new
    hws = dis * jnp.dot(h_new, wn_ref[...], preferred_element_type=jnp.float32)
    hao_ref[...] = hws[:, :HC]
    hbo_ref[...] = hws[:, HC:]


def _tc_layer(h, dis, s0, s1, ha, hb, b, g, bt, Wn):
    grid = (N // ROW_BLK,)
    rb = lambda i: (i, 0)
    z = lambda i: (0, 0)
    return pl.pallas_call(
        _tc_layer_body,
        grid=grid,
        in_specs=[
            pl.BlockSpec((ROW_BLK, H), rb),
            pl.BlockSpec((ROW_BLK, 1), rb),
            pl.BlockSpec((ROW_BLK, HC), rb),
            pl.BlockSpec((ROW_BLK, HC), rb),
            pl.BlockSpec((ROW_BLK, HC), rb),
            pl.BlockSpec((ROW_BLK, HC), rb),
            pl.BlockSpec((1, H), z),
            pl.BlockSpec((1, H), z),
            pl.BlockSpec((1, H), z),
            pl.BlockSpec((H, H), z),
        ],
        out_specs=[
            pl.BlockSpec((ROW_BLK, H), rb),
            pl.BlockSpec((ROW_BLK, HC), rb),
            pl.BlockSpec((ROW_BLK, HC), rb),
        ],
        out_shape=[
            jax.ShapeDtypeStruct((N, H), jnp.float32),
            jax.ShapeDtypeStruct((N, HC), jnp.float32),
            jax.ShapeDtypeStruct((N, HC), jnp.float32),
        ],
    )(h, dis, s0, s1, ha, hb, b, g, bt, Wn)


def _tc_final_body(h_ref, dis_ref, s0_ref, s1_ref, ha_ref, hb_ref,
                   b_ref, g_ref, bt_ref, batch_ref,
                   wo1_ref, bo1_ref, wo2_ref, bo2_ref,
                   out_ref, pooled_ref, cnt_ref):
    step = pl.program_id(0)
    nsteps = pl.num_programs(0)
    h_new, _ = _layer_update(h_ref, dis_ref, s0_ref, s1_ref, ha_ref,
                             hb_ref, b_ref, g_ref, bt_ref)
    bb = batch_ref[...]
    oh = (bb == lax.broadcasted_iota(jnp.int32, (1, G), 1)).astype(jnp.float32)
    dn = (((0,), (0,)), ((), ()))
    psum = lax.dot_general(oh, h_new, dn, preferred_element_type=jnp.float32)
    csum = lax.dot_general(oh, jnp.ones((oh.shape[0], 1), jnp.float32), dn,
                           preferred_element_type=jnp.float32)

    @pl.when(step == 0)
    def _():
        pooled_ref[...] = psum
        cnt_ref[...] = csum

    @pl.when(step > 0)
    def _():
        pooled_ref[...] += psum
        cnt_ref[...] += csum

    @pl.when(step == nsteps - 1)
    def _():
        pooled = pooled_ref[...] / jnp.maximum(cnt_ref[...], 1.0)
        t = jnp.maximum(
            jnp.dot(pooled, wo1_ref[...], preferred_element_type=jnp.float32)
            + bo1_ref[...], 0.0)
        out_ref[...] = (
            jnp.dot(t, wo2_ref[...], preferred_element_type=jnp.float32)
            + bo2_ref[...])


def _tc_final(h, dis, s0, s1, ha, hb, b, g, bt, batch2,
              Wo1, bo1, Wo2, bo2):
    grid = (N // ROW_BLK,)
    rb = lambda i: (i, 0)
    z = lambda i: (0, 0)
    return pl.pallas_call(
        _tc_final_body,
        grid=grid,
        in_specs=[
            pl.BlockSpec((ROW_BLK, H), rb),
            pl.BlockSpec((ROW_BLK, 1), rb),
            pl.BlockSpec((ROW_BLK, HC), rb),
            pl.BlockSpec((ROW_BLK, HC), rb),
            pl.BlockSpec((ROW_BLK, HC), rb),
            pl.BlockSpec((ROW_BLK, HC), rb),
            pl.BlockSpec((1, H), z),
            pl.BlockSpec((1, H), z),
            pl.BlockSpec((1, H), z),
            pl.BlockSpec((ROW_BLK, 1), rb),
            pl.BlockSpec((H, OUT), z),
            pl.BlockSpec((1, OUT), z),
            pl.BlockSpec((OUT, OUT), z),
            pl.BlockSpec((1, OUT), z),
        ],
        out_specs=pl.BlockSpec((G, OUT), z),
        out_shape=jax.ShapeDtypeStruct((G, OUT), jnp.float32),
        scratch_shapes=[
            pltpu.VMEM((G, H), jnp.float32),
            pltpu.VMEM((G, 1), jnp.float32),
        ],
    )(h, dis, s0, s1, ha, hb, b, g, bt, batch2, Wo1, bo1, Wo2, bo2)


def kernel(x, edge_index, batch, W_embed, b_embed, Ws, bs, gammas, betas,
           W_o1, b_o1, W_o2, b_o2):
    E = edge_index.shape[1]
    npad = E_PAD - E
    row = jnp.concatenate(
        [edge_index[0], jnp.zeros((npad,), jnp.int32)]).reshape(NB, B)
    col = jnp.concatenate(
        [edge_index[1], jnp.full((npad,), DUMMY_DST, jnp.int32)]).reshape(NB, B)

    d0, d1 = _sc_degree(col)
    h, dis, ha, hb = _tc_pre(x, d0, d1, W_embed,
                             b_embed.reshape(1, H), Ws[0])
    for l in range(L):
        s0, s1 = _sc_layer(row, col, ha, hb)
        if l < L - 1:
            h, ha, hb = _tc_layer(h, dis, s0, s1, ha, hb,
                                  bs[l].reshape(1, H),
                                  gammas[l].reshape(1, H),
                                  betas[l].reshape(1, H), Ws[l + 1])
        else:
            out = _tc_final(h, dis, s0, s1, ha, hb,
                            bs[l].reshape(1, H),
                            gammas[l].reshape(1, H),
                            betas[l].reshape(1, H),
                            batch.reshape(N, 1),
                            W_o1, b_o1.reshape(1, OUT),
                            W_o2, b_o2.reshape(1, OUT))
    return out


# speedup vs baseline: 15.1957x; 1.0104x over previous
import functools

import jax
import jax.numpy as jnp
from jax import lax
from jax.experimental import pallas as pl
from jax.experimental.pallas import tpu as pltpu
from jax.experimental.pallas import tpu_sc as plsc

N = 50000
H = 64
HC = 32
G = 16
OUT = 128
L = 4

NC = 2
NS = 16
B = 128
E_PAD = 819200
NB = E_PAD // B
S_ROWS = 51200
RT = S_ROWS // NS

DUMMY_DST = N

ROW_BLK = 2000


def _fill_f32(ref, rows, cols, val):
    v = jnp.full((16,), val, jnp.float32)

    def body(i, _):
        for c0 in range(0, cols, 16):
            ref[i, c0:c0 + 16] = v
        return 0

    lax.fori_loop(0, rows, body, 0)


def _sc_degree(col2):
    nbt = NB // (NC * NS)
    mesh = plsc.VectorSubcoreMesh(core_axis_name="c", subcore_axis_name="s")

    @functools.partial(
        pl.kernel,
        mesh=mesh,
        compiler_params=pltpu.CompilerParams(use_tc_tiling_on_sc=False),
        out_type=[
            jax.ShapeDtypeStruct((S_ROWS, 16), jnp.float32),
            jax.ShapeDtypeStruct((S_ROWS, 16), jnp.float32),
        ],
        scratch_types=[
            pltpu.VMEM((nbt, B), jnp.int32),
            pltpu.VMEM((B, 16), jnp.float32),
            pltpu.VMEM((B, 16), jnp.float32),
            pltpu.VMEM_SHARED((S_ROWS, 16), jnp.float32),
        ],
    )
    def k(col_hbm, d0_hbm, d1_hbm, cidx_v, ones_v, zero_v, deg_sh):
        cid = lax.axis_index("c")
        sid = lax.axis_index("s")
        wid = sid * NC + cid

        _fill_f32(ones_v, B, 16, 1.0)
        _fill_f32(zero_v, B, 16, 0.0)

        def zbody(j, _):
            pltpu.sync_copy(zero_v, deg_sh.at[pl.ds(sid * RT + j * B, B)])
            return 0
        lax.fori_loop(0, RT // B, zbody, 0)

        pltpu.sync_copy(col_hbm.at[pl.ds(wid * nbt, nbt)], cidx_v)

        plsc.subcore_barrier()

        def sbody(g, _):
            pltpu.sync_copy(ones_v, deg_sh.at[cidx_v.at[g]], add=True)
            return 0
        lax.fori_loop(0, nbt, sbody, 0)

        plsc.subcore_barrier()

        @pl.when(cid == 0)
        def _():
            pltpu.sync_copy(deg_sh.at[pl.ds(sid * RT, RT)],
                            d0_hbm.at[pl.ds(sid * RT, RT)])

        @pl.when(cid == 1)
        def _():
            pltpu.sync_copy(deg_sh.at[pl.ds(sid * RT, RT)],
                            d1_hbm.at[pl.ds(sid * RT, RT)])

    return k(col2)


def _sc_layer(row2, col2, hws_a, hws_b):
    nbt = NB // NS
    mesh = plsc.VectorSubcoreMesh(core_axis_name="c", subcore_axis_name="s")

    QB = 25
    NBUF = 5
    LOOK = 3

    @functools.partial(
        pl.kernel,
        mesh=mesh,
        compiler_params=pltpu.CompilerParams(use_tc_tiling_on_sc=False),
        out_type=[
            jax.ShapeDtypeStruct((S_ROWS, HC), jnp.float32),
            jax.ShapeDtypeStruct((S_ROWS, HC), jnp.float32),
        ],
        scratch_types=[
            pltpu.VMEM((QB, B), jnp.int32),
            pltpu.VMEM((QB, B), jnp.int32),
            pltpu.VMEM((NBUF * B, HC), jnp.float32),
            pltpu.VMEM_SHARED((S_ROWS, HC), jnp.float32),
        ] + [pltpu.SemaphoreType.DMA] * NBUF,
    )
    def k(row_hbm, col_hbm, ha_hbm, hb_hbm, s0_hbm, s1_hbm,
          ridx_v, cidx_v, rows_v, s_sh, *sems):
        cid = lax.axis_index("c")
        sid = lax.axis_index("s")

        _fill_f32(rows_v, B, HC, 0.0)

        def zbody(j, _):
            pltpu.sync_copy(rows_v.at[pl.ds(0, B)],
                            s_sh.at[pl.ds(sid * RT + j * B, B)])
            return 0
        lax.fori_loop(0, RT // B, zbody, 0)

        plsc.subcore_barrier()

        def run(tab_hbm):
            def buf(b):
                return rows_v.at[pl.ds(b * B, B)]

            def gather(j, b):
                pltpu.async_copy(tab_hbm.at[ridx_v.at[j]], buf(b), sems[b])

            def scatter(j, b):
                pltpu.async_copy(buf(b), s_sh.at[cidx_v.at[j]],
                                 sems[b], add=True)

            def wait(b):
                pltpu.make_async_copy(buf(b), s_sh.at[cidx_v.at[0]],
                                      sems[b]).wait()

            def slot(q, _):
                base = sid * nbt + q * QB
                pltpu.sync_copy(row_hbm.at[pl.ds(base, QB)], ridx_v)
                pltpu.sync_copy(col_hbm.at[pl.ds(base, QB)], cidx_v)
                for j in range(LOOK):
                    gather(j, j % NBUF)
                for j in range(QB):
                    b = j % NBUF
                    wait(b)
                    scatter(j, b)
                    jn = j + LOOK
                    if jn < QB:
                        b2 = jn % NBUF
                        if jn >= NBUF:
                            wait(b2)
                        gather(jn, b2)
                for j in range(QB - NBUF, QB):
                    wait(j % NBUF)
                return 0
            lax.fori_loop(0, nbt // QB, slot, 0)

        @pl.when(cid == 0)
        def _():
            run(ha_hbm)

        @pl.when(cid == 1)
        def _():
            run(hb_hbm)

        plsc.subcore_barrier()

        @pl.when(cid == 0)
        def _():
            pltpu.sync_copy(s_sh.at[pl.ds(sid * RT, RT)],
                            s0_hbm.at[pl.ds(sid * RT, RT)])

        @pl.when(cid == 1)
        def _():
            pltpu.sync_copy(s_sh.at[pl.ds(sid * RT, RT)],
                            s1_hbm.at[pl.ds(sid * RT, RT)])

    return k(row2, col2, hws_a, hws_b)


def _tc_pre_body(x_ref, d0_ref, d1_ref, we_ref, be_ref, w0_ref,
                 h_ref, dis_ref, ha_ref, hb_ref):
    xb = x_ref[...]
    h = jnp.maximum(
        jnp.dot(xb, we_ref[...], preferred_element_type=jnp.float32)
        + be_ref[...], 0.0)
    deg = d0_ref[:, 0:1] + d1_ref[:, 0:1] + 1.0
    dis = lax.rsqrt(deg)
    hws = dis * jnp.dot(h, w0_ref[...], preferred_element_type=jnp.float32)
    h_ref[...] = h
    dis_ref[...] = dis
    ha_ref[...] = hws[:, :HC]
    hb_ref[...] = hws[:, HC:]


def _tc_pre(x, d0, d1, We, be, W0):
    grid = (N // ROW_BLK,)
    return pl.pallas_call(
        _tc_pre_body,
        grid=grid,
        in_specs=[
            pl.BlockSpec((ROW_BLK, 9), lambda i: (i, 0)),
            pl.BlockSpec((ROW_BLK, 16), lambda i: (i, 0)),
            pl.BlockSpec((ROW_BLK, 16), lambda i: (i, 0)),
            pl.BlockSpec((9, H), lambda i: (0, 0)),
            pl.BlockSpec((1, H), lambda i: (0, 0)),
            pl.BlockSpec((H, H), lambda i: (0, 0)),
        ],
        out_specs=[
            pl.BlockSpec((ROW_BLK, H), lambda i: (i, 0)),
            pl.BlockSpec((ROW_BLK, 1), lambda i: (i, 0)),
            pl.BlockSpec((ROW_BLK, HC), lambda i: (i, 0)),
            pl.BlockSpec((ROW_BLK, HC), lambda i: (i, 0)),
        ],
        out_shape=[
            jax.ShapeDtypeStruct((N, H), jnp.float32),
            jax.ShapeDtypeStruct((N, 1), jnp.float32),
            jax.ShapeDtypeStruct((N, HC), jnp.float32),
            jax.ShapeDtypeStruct((N, HC), jnp.float32),
        ],
    )(x, d0, d1, We, be, W0)


def _layer_update(h_ref, dis_ref, s0_ref, s1_ref, ha_ref, hb_ref,
                  b_ref, g_ref, bt_ref):
    S = jnp.concatenate([s0_ref[...], s1_ref[...]], axis=1)
    hws = jnp.concatenate([ha_ref[...], hb_ref[...]], axis=1)
    dis = dis_ref[...]
    agg = dis * (S + hws) + b_ref[...]
    mu = jnp.mean(agg, axis=1, keepdims=True)
    diff = agg - mu
    var = jnp.mean(diff * diff, axis=1, keepdims=True)
    hn = diff * lax.rsqrt(var + 1e-5) * g_ref[...] + bt_ref[...]
    return h_ref[...] + jnp.maximum(hn, 0.0), dis


def _tc_layer_body(h_ref, dis_ref, s0_ref, s1_ref, ha_ref, hb_ref,
                   b_ref, g_ref, bt_ref, wn_ref,
                   ho_ref, hao_ref, hbo_ref):
    h_new, dis = _layer_update(h_ref, dis_ref, s0_ref, s1_ref, ha_ref,
                               hb_ref, b_ref, g_ref, bt_ref)
    ho_ref[...] = h_new
    hws = dis * jnp.dot(h_new, wn_ref[...], preferred_element_type=jnp.float32)
    hao_ref[...] = hws[:, :HC]
    hbo_ref[...] = hws[:, HC:]


def _tc_layer(h, dis, s0, s1, ha, hb, b, g, bt, Wn):
    grid = (N // ROW_BLK,)
    rb = lambda i: (i, 0)
    z = lambda i: (0, 0)
    return pl.pallas_call(
        _tc_layer_body,
        grid=grid,
        in_specs=[
            pl.BlockSpec((ROW_BLK, H), rb),
            pl.BlockSpec((ROW_BLK, 1), rb),
            pl.BlockSpec((ROW_BLK, HC), rb),
            pl.BlockSpec((ROW_BLK, HC), rb),
            pl.BlockSpec((ROW_BLK, HC), rb),
            pl.BlockSpec((ROW_BLK, HC), rb),
            pl.BlockSpec((1, H), z),
            pl.BlockSpec((1, H), z),
            pl.BlockSpec((1, H), z),
            pl.BlockSpec((H, H), z),
        ],
        out_specs=[
            pl.BlockSpec((ROW_BLK, H), rb),
            pl.BlockSpec((ROW_BLK, HC), rb),
            pl.BlockSpec((ROW_BLK, HC), rb),
        ],
        out_shape=[
            jax.ShapeDtypeStruct((N, H), jnp.float32),
            jax.ShapeDtypeStruct((N, HC), jnp.float32),
            jax.ShapeDtypeStruct((N, HC), jnp.float32),
        ],
    )(h, dis, s0, s1, ha, hb, b, g, bt, Wn)


def _tc_final_body(h_ref, dis_ref, s0_ref, s1_ref, ha_ref, hb_ref,
                   b_ref, g_ref, bt_ref, batch_ref,
                   wo1_ref, bo1_ref, wo2_ref, bo2_ref,
                   out_ref, pooled_ref, cnt_ref):
    step = pl.program_id(0)
    nsteps = pl.num_programs(0)
    h_new, _ = _layer_update(h_ref, dis_ref, s0_ref, s1_ref, ha_ref,
                             hb_ref, b_ref, g_ref, bt_ref)
    bb = batch_ref[...]
    oh = (bb == lax.broadcasted_iota(jnp.int32, (1, G), 1)).astype(jnp.float32)
    dn = (((0,), (0,)), ((), ()))
    psum = lax.dot_general(oh, h_new, dn, preferred_element_type=jnp.float32)
    csum = lax.dot_general(oh, jnp.ones((oh.shape[0], 1), jnp.float32), dn,
                           preferred_element_type=jnp.float32)

    @pl.when(step == 0)
    def _():
        pooled_ref[...] = psum
        cnt_ref[...] = csum

    @pl.when(step > 0)
    def _():
        pooled_ref[...] += psum
        cnt_ref[...] += csum

    @pl.when(step == nsteps - 1)
    def _():
        pooled = pooled_ref[...] / jnp.maximum(cnt_ref[...], 1.0)
        t = jnp.maximum(
            jnp.dot(pooled, wo1_ref[...], preferred_element_type=jnp.float32)
            + bo1_ref[...], 0.0)
        out_ref[...] = (
            jnp.dot(t, wo2_ref[...], preferred_element_type=jnp.float32)
            + bo2_ref[...])


def _tc_final(h, dis, s0, s1, ha, hb, b, g, bt, batch2,
              Wo1, bo1, Wo2, bo2):
    grid = (N // ROW_BLK,)
    rb = lambda i: (i, 0)
    z = lambda i: (0, 0)
    return pl.pallas_call(
        _tc_final_body,
        grid=grid,
        in_specs=[
            pl.BlockSpec((ROW_BLK, H), rb),
            pl.BlockSpec((ROW_BLK, 1), rb),
            pl.BlockSpec((ROW_BLK, HC), rb),
            pl.BlockSpec((ROW_BLK, HC), rb),
            pl.BlockSpec((ROW_BLK, HC), rb),
            pl.BlockSpec((ROW_BLK, HC), rb),
            pl.BlockSpec((1, H), z),
            pl.BlockSpec((1, H), z),
            pl.BlockSpec((1, H), z),
            pl.BlockSpec((ROW_BLK, 1), rb),
            pl.BlockSpec((H, OUT), z),
            pl.BlockSpec((1, OUT), z),
            pl.BlockSpec((OUT, OUT), z),
            pl.BlockSpec((1, OUT), z),
        ],
        out_specs=pl.BlockSpec((G, OUT), z),
        out_shape=jax.ShapeDtypeStruct((G, OUT), jnp.float32),
        scratch_shapes=[
            pltpu.VMEM((G, H), jnp.float32),
            pltpu.VMEM((G, 1), jnp.float32),
        ],
    )(h, dis, s0, s1, ha, hb, b, g, bt, batch2, Wo1, bo1, Wo2, bo2)


def kernel(x, edge_index, batch, W_embed, b_embed, Ws, bs, gammas, betas,
           W_o1, b_o1, W_o2, b_o2):
    E = edge_index.shape[1]
    npad = E_PAD - E
    row = jnp.concatenate(
        [edge_index[0], jnp.zeros((npad,), jnp.int32)]).reshape(NB, B)
    col = jnp.concatenate(
        [edge_index[1], jnp.full((npad,), DUMMY_DST, jnp.int32)]).reshape(NB, B)

    d0, d1 = _sc_degree(col)
    h, dis, ha, hb = _tc_pre(x, d0, d1, W_embed,
                             b_embed.reshape(1, H), Ws[0])
    for l in range(L):
        s0, s1 = _sc_layer(row, col, ha, hb)
        if l < L - 1:
            h, ha, hb = _tc_layer(h, dis, s0, s1, ha, hb,
                                  bs[l].reshape(1, H),
                                  gammas[l].reshape(1, H),
                                  betas[l].reshape(1, H), Ws[l + 1])
        else:
            out = _tc_final(h, dis, s0, s1, ha, hb,
                            bs[l].reshape(1, H),
                            gammas[l].reshape(1, H),
                            betas[l].reshape(1, H),
                            batch.reshape(N, 1),
                            W_o1, b_o1.reshape(1, OUT),
                            W_o2, b_o2.reshape(1, OUT))
    return out
